# baseline (speedup 1.0000x reference)
import functools

import jax
import jax.numpy as jnp
from jax import lax
from jax.experimental import pallas as pl
from jax.experimental.pallas import tpu as pltpu
from jax.experimental.pallas import tpu_sc as plsc

_NC = 2
_NS = 16
_NW = _NC * _NS
_C = 125
_ZR = 128


def _mesh():
    return plsc.VectorSubcoreMesh(core_axis_name="c", subcore_axis_name="s")


_SC_PARAMS = pltpu.CompilerParams(use_tc_tiling_on_sc=False)


@functools.lru_cache(None)
def _sc_degree(n_pad, e):
    epw = e // _NW
    nchunk = epw // _C
    rpt = n_pad // _NS

    @functools.partial(
        pl.kernel,
        mesh=_mesh(),
        out_type=jax.ShapeDtypeStruct((_NC, n_pad), jnp.float32),
        scratch_types=[
            pltpu.VMEM((nchunk, _C), jnp.int32),
            pltpu.VMEM((128,), jnp.float32),
            pltpu.VMEM((n_pad // _NS,), jnp.float32),
            pltpu.VMEM_SHARED((n_pad,), jnp.float32),
            pltpu.SemaphoreType.DMA,
            pltpu.SemaphoreType.DMA,
        ],
        compiler_params=_SC_PARAMS,
    )
    def deg_kernel(ei_hbm, out_hbm, dst_v, ones_v, zbuf, acc, isem, ssem):
        cid = lax.axis_index("c")
        sid = lax.axis_index("s")
        wid = sid * _NC + cid
        zv = jnp.zeros((16,), jnp.float32)
        ov = jnp.ones((16,), jnp.float32)

        idx_src = ei_hbm.at[1, pl.ds(wid * nchunk, nchunk)]
        pltpu.async_copy(idx_src, dst_v, isem)

        def fill_z(i, _):
            zbuf[pl.ds(i * 16, 16)] = zv
            return 0

        lax.fori_loop(0, rpt // 16, fill_z, 0)

        def fill_o(i, _):
            ones_v[pl.ds(i * 16, 16)] = ov
            return 0

        lax.fori_loop(0, 8, fill_o, 0)

        base_r = sid * rpt
        pltpu.sync_copy(zbuf, acc.at[pl.ds(base_r, rpt)])
        pltpu.make_async_copy(idx_src, dst_v, isem).wait()
        plsc.subcore_barrier()

        def body(i, _):
            pltpu.async_copy(ones_v.at[pl.ds(0, _C)], acc.at[dst_v.at[i]],
                             ssem, add=True)
            return 0

        lax.fori_loop(0, nchunk, body, 0)

        def drain(i, _):
            pltpu.make_async_copy(ones_v.at[pl.ds(0, _C)],
                                  acc.at[dst_v.at[i]], ssem).wait()
            return 0

        lax.fori_loop(0, nchunk, drain, 0)
        plsc.subcore_barrier()
        pltpu.sync_copy(acc.at[pl.ds(base_r, rpt)],
                        out_hbm.at[cid, pl.ds(base_r, rpt)])

    return deg_kernel


@functools.lru_cache(None)
def _sc_edge_pass(n_pad, e, f):
    epw = e // _NW
    nchunk = epw // _C
    rpt = n_pad // _NS
    nb = 2 if f >= 128 else 5
    k_slack = nb // 2
    dst_bulk = f < 128
    nfull = nchunk // nb
    ntail = nchunk - nfull * nb

    @functools.partial(
        pl.kernel,
        mesh=_mesh(),
        out_type=jax.ShapeDtypeStruct((_NC, n_pad, f), jnp.float32),
        scratch_types=[
            pltpu.VMEM((nchunk, _C), jnp.int32),
            pltpu.VMEM((nchunk if dst_bulk else nb, _C), jnp.int32),
            pltpu.VMEM((nb * _C, f), jnp.float32),
            pltpu.VMEM_SHARED((n_pad, f), jnp.float32),
            pltpu.SemaphoreType.DMA,
            pltpu.SemaphoreType.DMA,
        ] + [pltpu.SemaphoreType.DMA] * (3 * nb),
        compiler_params=_SC_PARAMS,
    )
    def edge_kernel(h_hbm, ei_hbm, out_hbm,
                    src_v, dst_v, rows_v, acc, isem0, isem1, *sems):
        gsems = sems[:nb]
        dsems = sems[nb:2 * nb]
        ssems = sems[2 * nb:]
        cid = lax.axis_index("c")
        sid = lax.axis_index("s")
        wid = sid * _NC + cid
        zv = jnp.zeros((16,), jnp.float32)
        row0 = wid * nchunk

        src_rows = ei_hbm.at[0, pl.ds(row0, nchunk)]
        pltpu.async_copy(src_rows, src_v, isem0)
        if dst_bulk:
            dst_rows = ei_hbm.at[1, pl.ds(row0, nchunk)]
            pltpu.async_copy(dst_rows, dst_v, isem1)

        def fill_z(i, _):
            for j in range(f // 16):
                rows_v[i, pl.ds(j * 16, 16)] = zv
            return 0

        lax.fori_loop(0, _ZR, fill_z, 0)
        base_r = sid * rpt
        zval = rows_v.at[pl.ds(0, _ZR)]
        for k in range(rpt // _ZR):
            pltpu.sync_copy(zval, acc.at[pl.ds(base_r + k * _ZR, _ZR)])
        pltpu.make_async_copy(src_rows, src_v, isem0).wait()
        if dst_bulk:
            pltpu.make_async_copy(dst_rows, dst_v, isem1).wait()
        plsc.subcore_barrier()

        def gather_start(i, b):
            pltpu.async_copy(h_hbm.at[src_v.at[i]],
                             rows_v.at[pl.ds(b * _C, _C)], gsems[b])

        def gather_wait(i, b):
            pltpu.make_async_copy(h_hbm.at[src_v.at[i]],
                                  rows_v.at[pl.ds(b * _C, _C)],
                                  gsems[b]).wait()

        def didx_start(i, b):
            if not dst_bulk:
                pltpu.async_copy(ei_hbm.at[1, pl.ds(row0 + i, 1)],
                                 dst_v.at[pl.ds(b, 1)], dsems[b])

        def didx_wait(i, b):
            if not dst_bulk:
                pltpu.make_async_copy(ei_hbm.at[1, pl.ds(row0 + i, 1)],
                                      dst_v.at[pl.ds(b, 1)], dsems[b]).wait()

        def scatter_start(i, b):
            idx = dst_v.at[i] if dst_bulk else dst_v.at[b]
            pltpu.async_copy(rows_v.at[pl.ds(b * _C, _C)],
                             acc.at[idx], ssems[b], add=True)

        def scatter_wait(i, b):
            idx = dst_v.at[i] if dst_bulk else dst_v.at[b]
            pltpu.make_async_copy(rows_v.at[pl.ds(b * _C, _C)],
                                  acc.at[idx], ssems[b]).wait()

        for b in range(nb):
            didx_start(b, b)
            gather_start(b, b)

        def group(g, _):
            for b in range(nb):
                i = g * nb + b
                gather_wait(i, b)
                didx_wait(i, b)
                scatter_start(i, b)
                j = i - k_slack
                b2 = (b - k_slack) % nb

                @pl.when((j >= 0) & (j + nb < nchunk))
                def _refill():
                    scatter_wait(j, b2)
                    didx_start(j + nb, b2)
                    gather_start(j + nb, b2)

            return 0

        lax.fori_loop(0, nfull, group, 0)
        for b in range(ntail):
            i = nfull * nb + b
            gather_wait(i, b)
            didx_wait(i, b)
            scatter_start(i, b)

        drain_from = min(nfull * nb - k_slack, nchunk - nb)
        for c in range(max(0, drain_from), nchunk):
            scatter_wait(c, c % nb)

        plsc.subcore_barrier()
        pltpu.sync_copy(acc.at[pl.ds(base_r, rpt)],
                        out_hbm.at[cid, pl.ds(base_r, rpt)])

    return edge_kernel


def _tc_pre(x, w, degp):
    n, _ = x.shape
    n_pad = degp.shape[1]
    f = w.shape[1]

    def body(x_ref, w_ref, degp_ref, h_ref, dinv_ref):
        deg = degp_ref[0:1, :] + degp_ref[1:2, :] + 1.0
        dinv_row = 1.0 / jnp.sqrt(deg)
        dinv_ref[...] = dinv_row
        dinv = jnp.transpose(dinv_row, (1, 0))[:n]
        h_ref[...] = jnp.dot(x_ref[...], w_ref[...],
                             preferred_element_type=jnp.float32) * dinv

    return pl.pallas_call(
        body,
        out_shape=(jax.ShapeDtypeStruct((n, f), jnp.float32),
                   jax.ShapeDtypeStruct((1, n_pad), jnp.float32)),
    )(x, w, degp)


def _tc_mid(sp, hp, dinv, b, w):
    n, f = hp.shape
    f_next = w.shape[1]

    def body(sp_ref, h_ref, dinv_ref, b_ref, w_ref, out_ref):
        s = sp_ref[0, :n, :] + sp_ref[1, :n, :]
        di = jnp.transpose(dinv_ref[...], (1, 0))[:n]
        t = (s + h_ref[...]) * di + b_ref[...]
        x2 = jnp.maximum(t, 0.0)
        out_ref[...] = jnp.dot(x2, w_ref[...],
                               preferred_element_type=jnp.float32) * di

    return pl.pallas_call(
        body,
        out_shape=jax.ShapeDtypeStruct((n, f_next), jnp.float32),
    )(sp, hp, dinv, b, w)


def _tc_final(sp, hp, dinv, b, watt, fcw, fcb, sw, sb):
    n, f = hp.shape

    def body(sp_ref, h_ref, dinv_ref, b_ref, watt_ref, fcw_ref, fcb_ref,
             sw_ref, sb_ref, out_ref):
        s = sp_ref[0, :n, :] + sp_ref[1, :n, :]
        di = jnp.transpose(dinv_ref[...], (1, 0))[:n]
        h = (s + h_ref[...]) * di + b_ref[...]
        hw = jnp.dot(h, watt_ref[...], preferred_element_type=jnp.float32)
        gc = jnp.sum(hw, axis=0, keepdims=True) * (1.0 / n)
        tg = jnp.tanh(gc)
        scores = jax.nn.sigmoid(jnp.sum(h * tg, axis=1, keepdims=True))
        rep = jnp.sum(h * scores, axis=0, keepdims=True)
        t1 = jnp.dot(rep, fcw_ref[...], preferred_element_type=jnp.float32)
        t1 = jnp.maximum(t1 + fcb_ref[...], 0.0)
        t2 = jnp.dot(t1, sw_ref[...], preferred_element_type=jnp.float32)
        out_ref[...] = jax.nn.sigmoid(t2 + sb_ref[...])

    return pl.pallas_call(
        body,
        out_shape=jax.ShapeDtypeStruct((1, 1), jnp.float32),
    )(sp, hp, dinv, b, watt, fcw, fcb, sw, sb)


def kernel(features_1, edge_index_1, W1, b1, W2, b2, W3, b3, Watt, fcW, fcb,
           sW, sb):
    n, _ = features_1.shape
    e = edge_index_1.shape[1]
    assert e % (_NW * _C) == 0, "edge count must tile over 32 subcores x 80"
    quantum = _NS * _ZR
    n_pad = ((n + quantum - 1) // quantum) * quantum

    ei3 = edge_index_1.astype(jnp.int32).reshape(2, e // _C, _C)

    degp = _sc_degree(n_pad, e)(ei3)
    h1p, dinv = _tc_pre(features_1, W1, degp)
    s1 = _sc_edge_pass(n_pad, e, W1.shape[1])(h1p, ei3)
    h2p = _tc_mid(s1, h1p, dinv, b1.reshape(1, -1), W2)
    s2 = _sc_edge_pass(n_pad, e, W2.shape[1])(h2p, ei3)
    h3p = _tc_mid(s2, h2p, dinv, b2.reshape(1, -1), W3)
    s3 = _sc_edge_pass(n_pad, e, W3.shape[1])(h3p, ei3)
    return _tc_final(s3, h3p, dinv, b3.reshape(1, -1), Watt, fcW,
                     fcb.reshape(1, -1), sW, sb.reshape(1, -1))


# speedup vs baseline: 1.1039x; 1.1039x over previous
import functools

import jax
import jax.numpy as jnp
from jax import lax
from jax.experimental import pallas as pl
from jax.experimental.pallas import tpu as pltpu
from jax.experimental.pallas import tpu_sc as plsc

_NC = 2
_NS = 16
_NW = _NC * _NS
_C = 125
_ZR = 128


def _mesh():
    return plsc.VectorSubcoreMesh(core_axis_name="c", subcore_axis_name="s")


_SC_PARAMS = pltpu.CompilerParams(use_tc_tiling_on_sc=False)


@functools.lru_cache(None)
def _sc_degree(n_pad, e):
    epw = e // _NW
    nchunk = epw // _C
    rpt = n_pad // _NS

    @functools.partial(
        pl.kernel,
        mesh=_mesh(),
        out_type=jax.ShapeDtypeStruct((_NC, n_pad), jnp.float32),
        scratch_types=[
            pltpu.VMEM((nchunk, _C), jnp.int32),
            pltpu.VMEM((128,), jnp.float32),
            pltpu.VMEM((n_pad // _NS,), jnp.float32),
            pltpu.VMEM_SHARED((n_pad,), jnp.float32),
            pltpu.SemaphoreType.DMA,
            pltpu.SemaphoreType.DMA,
        ],
        compiler_params=_SC_PARAMS,
    )
    def deg_kernel(ei_hbm, out_hbm, dst_v, ones_v, zbuf, acc, isem, ssem):
        cid = lax.axis_index("c")
        sid = lax.axis_index("s")
        wid = sid * _NC + cid
        zv = jnp.zeros((16,), jnp.float32)
        ov = jnp.ones((16,), jnp.float32)

        idx_src = ei_hbm.at[1, pl.ds(wid * nchunk, nchunk)]
        pltpu.async_copy(idx_src, dst_v, isem)

        def fill_z(i, _):
            zbuf[pl.ds(i * 16, 16)] = zv
            return 0

        lax.fori_loop(0, rpt // 16, fill_z, 0)

        def fill_o(i, _):
            ones_v[pl.ds(i * 16, 16)] = ov
            return 0

        lax.fori_loop(0, 8, fill_o, 0)

        base_r = sid * rpt
        pltpu.sync_copy(zbuf, acc.at[pl.ds(base_r, rpt)])
        pltpu.make_async_copy(idx_src, dst_v, isem).wait()
        plsc.subcore_barrier()

        def body(i, _):
            pltpu.async_copy(ones_v.at[pl.ds(0, _C)], acc.at[dst_v.at[i]],
                             ssem, add=True)
            return 0

        lax.fori_loop(0, nchunk, body, 0)

        def drain(i, _):
            pltpu.make_async_copy(ones_v.at[pl.ds(0, _C)],
                                  acc.at[dst_v.at[i]], ssem).wait()
            return 0

        lax.fori_loop(0, nchunk, drain, 0)
        plsc.subcore_barrier()
        pltpu.sync_copy(acc.at[pl.ds(base_r, rpt)],
                        out_hbm.at[cid, pl.ds(base_r, rpt)])

    return deg_kernel


@functools.lru_cache(None)
def _sc_edge_pass(n_pad, e, f):
    epw = e // _NW
    nchunk = epw // _C
    rpt = n_pad // _NS
    nb = 2 if f >= 128 else 5
    dst_bulk = f < 128
    nfull = nchunk // nb
    ntail = nchunk - nfull * nb

    @functools.partial(
        pl.kernel,
        mesh=_mesh(),
        out_type=jax.ShapeDtypeStruct((_NC, n_pad, f), jnp.float32),
        scratch_types=[
            pltpu.VMEM((nchunk, _C), jnp.int32),
            pltpu.VMEM((nchunk if dst_bulk else nb, _C), jnp.int32),
            pltpu.VMEM((nb * _C, f), jnp.float32),
            pltpu.VMEM_SHARED((n_pad, f), jnp.float32),
            pltpu.SemaphoreType.DMA,
            pltpu.SemaphoreType.DMA,
        ] + [pltpu.SemaphoreType.DMA] * (2 * nb),
        compiler_params=_SC_PARAMS,
    )
    def edge_kernel(h_hbm, ei_hbm, out_hbm,
                    src_v, dst_v, rows_v, acc, isem0, isem1, *sems):
        gsems = sems[:nb]
        dsems = sems[nb:2 * nb]
        cid = lax.axis_index("c")
        sid = lax.axis_index("s")
        wid = sid * _NC + cid
        zv = jnp.zeros((16,), jnp.float32)
        row0 = wid * nchunk

        src_rows = ei_hbm.at[0, pl.ds(row0, nchunk)]
        pltpu.async_copy(src_rows, src_v, isem0)
        if dst_bulk:
            dst_rows = ei_hbm.at[1, pl.ds(row0, nchunk)]
            pltpu.async_copy(dst_rows, dst_v, isem1)

        def fill_z(i, _):
            for j in range(f // 16):
                rows_v[i, pl.ds(j * 16, 16)] = zv
            return 0

        lax.fori_loop(0, _ZR, fill_z, 0)
        base_r = sid * rpt
        zval = rows_v.at[pl.ds(0, _ZR)]
        for k in range(rpt // _ZR):
            pltpu.sync_copy(zval, acc.at[pl.ds(base_r + k * _ZR, _ZR)])
        pltpu.make_async_copy(src_rows, src_v, isem0).wait()
        if dst_bulk:
            pltpu.make_async_copy(dst_rows, dst_v, isem1).wait()
        plsc.subcore_barrier()

        def gather_start(i, b):
            pltpu.async_copy(h_hbm.at[src_v.at[i]],
                             rows_v.at[pl.ds(b * _C, _C)], gsems[b])

        def gather_wait(i, b):
            pltpu.make_async_copy(h_hbm.at[src_v.at[i]],
                                  rows_v.at[pl.ds(b * _C, _C)],
                                  gsems[b]).wait()

        def didx_start(i, b):
            if not dst_bulk:
                pltpu.async_copy(ei_hbm.at[1, pl.ds(row0 + i, 1)],
                                 dst_v.at[pl.ds(b, 1)], dsems[b])

        def didx_wait(i, b):
            if not dst_bulk:
                pltpu.make_async_copy(ei_hbm.at[1, pl.ds(row0 + i, 1)],
                                      dst_v.at[pl.ds(b, 1)], dsems[b]).wait()

        def scatter(i, b):
            idx = dst_v.at[i] if dst_bulk else dst_v.at[b]
            pltpu.sync_copy(rows_v.at[pl.ds(b * _C, _C)],
                            acc.at[idx], add=True)

        for b in range(nb):
            didx_start(b, b)
            gather_start(b, b)

        def group(g, _):
            for b in range(nb):
                i = g * nb + b
                gather_wait(i, b)
                didx_wait(i, b)
                scatter(i, b)

                @pl.when(i + nb < nchunk)
                def _refill():
                    didx_start(i + nb, b)
                    gather_start(i + nb, b)

            return 0

        lax.fori_loop(0, nfull, group, 0)
        for b in range(ntail):
            i = nfull * nb + b
            gather_wait(i, b)
            didx_wait(i, b)
            scatter(i, b)

        plsc.subcore_barrier()
        pltpu.sync_copy(acc.at[pl.ds(base_r, rpt)],
                        out_hbm.at[cid, pl.ds(base_r, rpt)])

    return edge_kernel


def _tc_pre(x, w, degp):
    n, _ = x.shape
    n_pad = degp.shape[1]
    f = w.shape[1]

    def body(x_ref, w_ref, degp_ref, h_ref, dinv_ref):
        deg = degp_ref[0:1, :] + degp_ref[1:2, :] + 1.0
        dinv_row = 1.0 / jnp.sqrt(deg)
        dinv_ref[...] = dinv_row
        dinv = jnp.transpose(dinv_row, (1, 0))[:n]
        h_ref[...] = jnp.dot(x_ref[...], w_ref[...],
                             preferred_element_type=jnp.float32) * dinv

    return pl.pallas_call(
        body,
        out_shape=(jax.ShapeDtypeStruct((n, f), jnp.float32),
                   jax.ShapeDtypeStruct((1, n_pad), jnp.float32)),
    )(x, w, degp)


def _tc_mid(sp, hp, dinv, b, w):
    n, f = hp.shape
    f_next = w.shape[1]

    def body(sp_ref, h_ref, dinv_ref, b_ref, w_ref, out_ref):
        s = sp_ref[0, :n, :] + sp_ref[1, :n, :]
        di = jnp.transpose(dinv_ref[...], (1, 0))[:n]
        t = (s + h_ref[...]) * di + b_ref[...]
        x2 = jnp.maximum(t, 0.0)
        out_ref[...] = jnp.dot(x2, w_ref[...],
                               preferred_element_type=jnp.float32) * di

    return pl.pallas_call(
        body,
        out_shape=jax.ShapeDtypeStruct((n, f_next), jnp.float32),
    )(sp, hp, dinv, b, w)


def _tc_final(sp, hp, dinv, b, watt, fcw, fcb, sw, sb):
    n, f = hp.shape

    def body(sp_ref, h_ref, dinv_ref, b_ref, watt_ref, fcw_ref, fcb_ref,
             sw_ref, sb_ref, out_ref):
        s = sp_ref[0, :n, :] + sp_ref[1, :n, :]
        di = jnp.transpose(dinv_ref[...], (1, 0))[:n]
        h = (s + h_ref[...]) * di + b_ref[...]
        hw = jnp.dot(h, watt_ref[...], preferred_element_type=jnp.float32)
        gc = jnp.sum(hw, axis=0, keepdims=True) * (1.0 / n)
        tg = jnp.tanh(gc)
        scores = jax.nn.sigmoid(jnp.sum(h * tg, axis=1, keepdims=True))
        rep = jnp.sum(h * scores, axis=0, keepdims=True)
        t1 = jnp.dot(rep, fcw_ref[...], preferred_element_type=jnp.float32)
        t1 = jnp.maximum(t1 + fcb_ref[...], 0.0)
        t2 = jnp.dot(t1, sw_ref[...], preferred_element_type=jnp.float32)
        out_ref[...] = jax.nn.sigmoid(t2 + sb_ref[...])

    return pl.pallas_call(
        body,
        out_shape=jax.ShapeDtypeStruct((1, 1), jnp.float32),
    )(sp, hp, dinv, b, watt, fcw, fcb, sw, sb)


def kernel(features_1, edge_index_1, W1, b1, W2, b2, W3, b3, Watt, fcW, fcb,
           sW, sb):
    n, _ = features_1.shape
    e = edge_index_1.shape[1]
    assert e % (_NW * _C) == 0, "edge count must tile over 32 subcores x 80"
    quantum = _NS * _ZR
    n_pad = ((n + quantum - 1) // quantum) * quantum

    ei3 = edge_index_1.astype(jnp.int32).reshape(2, e // _C, _C)

    degp = _sc_degree(n_pad, e)(ei3)
    h1p, dinv = _tc_pre(features_1, W1, degp)
    s1 = _sc_edge_pass(n_pad, e, W1.shape[1])(h1p, ei3)
    h2p = _tc_mid(s1, h1p, dinv, b1.reshape(1, -1), W2)
    s2 = _sc_edge_pass(n_pad, e, W2.shape[1])(h2p, ei3)
    h3p = _tc_mid(s2, h2p, dinv, b2.reshape(1, -1), W3)
    s3 = _sc_edge_pass(n_pad, e, W3.shape[1])(h3p, ei3)
    return _tc_final(s3, h3p, dinv, b3.reshape(1, -1), Watt, fcW,
                     fcb.reshape(1, -1), sW, sb.reshape(1, -1))
